# Initial kernel scaffold; baseline (speedup 1.0000x reference)
import jax
import jax.numpy as jnp
from jax.experimental import pallas as pl


def kernel(x, router_weight, gate_w, up_w, down_w, shared_gate_w, shared_up_w, shared_down_w):
    raise NotImplementedError("write your pallas kernel here")


# speedup vs baseline: 2.3566x; 2.3566x over previous
import functools

import jax
import jax.numpy as jnp
from jax import lax
from jax.experimental import pallas as pl
from jax.experimental.pallas import tpu as pltpu
from jax.experimental.pallas import tpu_sc as plsc

T = 2048
H = 2048
I = 1536
E = 64
TOPK = 4
SCALE = 1.8

TM = 128
IC = 2
ICH = I // IC
P = T * TOPK + E * TM
G = P // TM
NC, NS = 2, 16
NW = NC * NS
ROWS_W = P // NW
CH = 32
TOK_W = T // NW
CHT = 8


def _router_body(x_ref, rw_ref, sel_ref, wts_ref):
    x = x_ref[...]
    rw = rw_ref[...]
    logits = lax.dot_general(x, rw, (((1,), (1,)), ((), ())),
                             preferred_element_type=jnp.float32)
    bm = logits.shape[0]
    iota = lax.broadcasted_iota(jnp.int32, (bm, E), 1)
    vals, idxs = [], []
    l = logits
    for _ in range(TOPK):
        m = jnp.max(l, axis=1, keepdims=True)
        am = jnp.min(jnp.where(l == m, iota, jnp.int32(2**30)),
                     axis=1, keepdims=True)
        vals.append(m)
        idxs.append(am)
        l = jnp.where(iota == am, -jnp.inf, l)
    es = [jnp.exp(v - vals[0]) for v in vals]
    s = es[0] + es[1] + es[2] + es[3]
    ws = [e / s * SCALE for e in es]
    lane = lax.broadcasted_iota(jnp.int32, (bm, 128), 1)
    sel = jnp.zeros((bm, 128), jnp.int32)
    wts = jnp.zeros((bm, 128), jnp.float32)
    for k in range(TOPK):
        sel = jnp.where(lane == k, idxs[k], sel)
        wts = jnp.where(lane == k, ws[k], wts)
    sel_ref[...] = sel
    wts_ref[...] = wts


def _router(x_flat, router_weight):
    bm = 256
    return pl.pallas_call(
        _router_body,
        grid=(T // bm,),
        in_specs=[
            pl.BlockSpec((bm, H), lambda i: (i, 0)),
            pl.BlockSpec((E, H), lambda i: (0, 0)),
        ],
        out_specs=[
            pl.BlockSpec((bm, 128), lambda i: (i, 0)),
            pl.BlockSpec((bm, 128), lambda i: (i, 0)),
        ],
        out_shape=[
            jax.ShapeDtypeStruct((T, 128), jnp.int32),
            jax.ShapeDtypeStruct((T, 128), jnp.float32),
        ],
    )(x_flat, router_weight)


@functools.lru_cache(maxsize=1)
def _sc_gather_fn():
    mesh = plsc.VectorSubcoreMesh(core_axis_name="c", subcore_axis_name="s")

    @functools.partial(
        pl.kernel,
        out_type=jax.ShapeDtypeStruct((P, H), jnp.float32),
        mesh=mesh,
        scratch_types=[
            pltpu.VMEM((CH,), jnp.int32),
            pltpu.VMEM((CH, H), jnp.float32),
            pltpu.SemaphoreType.DMA,
        ],
    )
    def body(idx_hbm, x_hbm, out_hbm, idx_v, rows_v, sem):
        wid = lax.axis_index("s") * NC + lax.axis_index("c")
        base = wid * ROWS_W
        for c in range(ROWS_W // CH):
            off = base + c * CH
            pltpu.sync_copy(idx_hbm.at[pl.ds(off, CH)], idx_v)
            pltpu.async_copy(x_hbm.at[idx_v], rows_v, sem).wait()
            pltpu.sync_copy(rows_v, out_hbm.at[pl.ds(off, CH)])

    return body


def _sc_gather(tokens_padded, x_flat):
    return _sc_gather_fn()(tokens_padded, x_flat)


def _gemm_body(te_ref, used_ref, xs_ref, gw_ref, uw_ref, dw_ref, wp_ref,
               eo_ref):
    g = pl.program_id(0)
    c = pl.program_id(1)

    @pl.when(g < used_ref[0])
    def _():
        xb = xs_ref[...].astype(jnp.bfloat16)
        gw = gw_ref[0].astype(jnp.bfloat16)
        uw = uw_ref[0].astype(jnp.bfloat16)
        gate = lax.dot_general(xb, gw, (((1,), (1,)), ((), ())),
                               preferred_element_type=jnp.float32)
        up = lax.dot_general(xb, uw, (((1,), (1,)), ((), ())),
                             preferred_element_type=jnp.float32)
        h = (gate / (1.0 + jnp.exp(-gate))) * up
        hb = h.astype(jnp.bfloat16)
        dw = dw_ref[0].astype(jnp.bfloat16)
        part = lax.dot_general(hb, dw, (((1,), (1,)), ((), ())),
                               preferred_element_type=jnp.float32)
        part = part * wp_ref[...]

        @pl.when(c == 0)
        def _():
            eo_ref[...] = part

        @pl.when(c != 0)
        def _():
            eo_ref[...] += part


def _grouped_gemm(tile_expert, used, xs, gate_w, up_w, down_w, wp2):
    grid_spec = pltpu.PrefetchScalarGridSpec(
        num_scalar_prefetch=2,
        grid=(G, IC),
        in_specs=[
            pl.BlockSpec((TM, H), lambda g, c, te, u: (g, 0)),
            pl.BlockSpec((1, ICH, H), lambda g, c, te, u: (te[g], c, 0)),
            pl.BlockSpec((1, ICH, H), lambda g, c, te, u: (te[g], c, 0)),
            pl.BlockSpec((1, H, ICH), lambda g, c, te, u: (te[g], 0, c)),
            pl.BlockSpec((TM, 1), lambda g, c, te, u: (g, 0)),
        ],
        out_specs=pl.BlockSpec((TM, H), lambda g, c, te, u: (g, 0)),
    )
    return pl.pallas_call(
        _gemm_body,
        grid_spec=grid_spec,
        out_shape=jax.ShapeDtypeStruct((P, H), jnp.float32),
        compiler_params=pltpu.CompilerParams(
            dimension_semantics=("arbitrary", "arbitrary"),
            vmem_limit_bytes=100 * 1024 * 1024,
        ),
    )(tile_expert, used, xs, gate_w, up_w, down_w, wp2)


def _shared_body(x_ref, gw_ref, uw_ref, dw_ref, out_ref):
    xb = x_ref[...].astype(jnp.bfloat16)
    gw = gw_ref[...].astype(jnp.bfloat16)
    uw = uw_ref[...].astype(jnp.bfloat16)
    gate = lax.dot_general(xb, gw, (((1,), (1,)), ((), ())),
                           preferred_element_type=jnp.float32)
    up = lax.dot_general(xb, uw, (((1,), (1,)), ((), ())),
                         preferred_element_type=jnp.float32)
    h = (gate / (1.0 + jnp.exp(-gate))) * up
    hb = h.astype(jnp.bfloat16)
    dw = dw_ref[...].astype(jnp.bfloat16)
    out_ref[...] = lax.dot_general(hb, dw, (((1,), (1,)), ((), ())),
                                   preferred_element_type=jnp.float32)


def _shared_expert(x_flat, sgw, suw, sdw):
    bm = 256
    return pl.pallas_call(
        _shared_body,
        grid=(T // bm,),
        in_specs=[
            pl.BlockSpec((bm, H), lambda i: (i, 0)),
            pl.BlockSpec((I, H), lambda i: (0, 0)),
            pl.BlockSpec((I, H), lambda i: (0, 0)),
            pl.BlockSpec((H, I), lambda i: (0, 0)),
        ],
        out_specs=pl.BlockSpec((bm, H), lambda i: (i, 0)),
        out_shape=jax.ShapeDtypeStruct((T, H), jnp.float32),
        compiler_params=pltpu.CompilerParams(
            vmem_limit_bytes=100 * 1024 * 1024,
        ),
    )(x_flat, sgw, suw, sdw)


@functools.lru_cache(maxsize=1)
def _sc_combine_fn():
    mesh = plsc.VectorSubcoreMesh(core_axis_name="c", subcore_axis_name="s")

    @functools.partial(
        pl.kernel,
        out_type=jax.ShapeDtypeStruct((T, H), jnp.float32),
        mesh=mesh,
        scratch_types=[
            pltpu.VMEM((TOPK * CHT,), jnp.int32),
            pltpu.VMEM((TOPK * CHT, H), jnp.float32),
            pltpu.VMEM((CHT, H), jnp.float32),
            pltpu.VMEM((CHT, H), jnp.float32),
            pltpu.SemaphoreType.DMA,
        ],
    )
    def body(cidx_hbm, eo_hbm, sh_hbm, out_hbm, idx_v, rows_v, sh_v, ob_v,
             sem):
        wid = lax.axis_index("s") * NC + lax.axis_index("c")
        tbase = wid * TOK_W
        for cch in range(TOK_W // CHT):
            tok = tbase + cch * CHT
            pltpu.sync_copy(cidx_hbm.at[pl.ds(tok * TOPK, CHT * TOPK)], idx_v)
            pltpu.async_copy(eo_hbm.at[idx_v], rows_v, sem).wait()
            pltpu.sync_copy(sh_hbm.at[pl.ds(tok, CHT)], sh_v)

            def inner(i, _):
                cc = i * 16
                for j in range(CHT):
                    acc = sh_v[j, pl.ds(cc, 16)]
                    for k in range(TOPK):
                        acc = acc + rows_v[TOPK * j + k, pl.ds(cc, 16)]
                    ob_v[j, pl.ds(cc, 16)] = acc
                return 0

            lax.fori_loop(0, H // 16, inner, 0)
            pltpu.sync_copy(ob_v, out_hbm.at[pl.ds(tok, CHT)])

    return body


def _sc_combine(pos, eo, shared_out):
    return _sc_combine_fn()(pos, eo, shared_out)


def kernel(x, router_weight, gate_w, up_w, down_w, shared_gate_w,
           shared_up_w, shared_down_w):
    b, s, h = x.shape
    x_flat = x.astype(jnp.float32).reshape(-1, h)

    sel128, wts128 = _router(x_flat, router_weight)
    sel = sel128[:, :TOPK]
    wts = wts128[:, :TOPK]

    e = sel.reshape(-1)
    onehot = (e[:, None] == jnp.arange(E, dtype=e.dtype)[None, :])
    cum = jnp.cumsum(onehot.astype(jnp.int32), axis=0)
    rank = jnp.take_along_axis(cum, e[:, None].astype(jnp.int32),
                               axis=1)[:, 0] - 1
    counts = cum[-1]
    padded_counts = ((counts + TM - 1) // TM) * TM
    padded_off = jnp.concatenate(
        [jnp.zeros((1,), jnp.int32), jnp.cumsum(padded_counts)[:-1]])
    pos = padded_off[e] + rank
    tokens = jnp.arange(T * TOPK, dtype=jnp.int32) // TOPK
    tokens_padded = jnp.zeros((P,), jnp.int32).at[pos].set(tokens)
    wp = jnp.zeros((P,), jnp.float32).at[pos].set(wts.reshape(-1))
    start_tile = padded_off // TM
    used_tiles = ((padded_off[-1] + padded_counts[-1]) // TM).astype(jnp.int32)
    tile_expert = (jnp.searchsorted(start_tile,
                                    jnp.arange(G, dtype=jnp.int32),
                                    side="right") - 1).astype(jnp.int32)

    xs = _sc_gather(tokens_padded, x_flat)
    eo = _grouped_gemm(tile_expert, used_tiles.reshape(1), xs,
                       gate_w, up_w, down_w, wp.reshape(P, 1))
    shared_out = _shared_expert(x_flat, shared_gate_w, shared_up_w,
                                shared_down_w)
    out = _sc_combine(pos.astype(jnp.int32), eo, shared_out)
    return out.reshape(b, s, h)
